# Initial kernel scaffold; baseline (speedup 1.0000x reference)
import jax
import jax.numpy as jnp
from jax.experimental import pallas as pl


def kernel(x, table):
    raise NotImplementedError("write your pallas kernel here")


# speedup vs baseline: 2.0450x; 2.0450x over previous
import functools

import jax
import jax.numpy as jnp
from jax import lax
from jax.experimental import pallas as pl
from jax.experimental.pallas import tpu as pltpu
from jax.experimental.pallas import tpu_sc as plsc

MOD = 1000000
EMBED = 32
NUM_WORKERS = 32
GATHER = 128
KB = 8
BATCH = GATHER * KB


@functools.partial(jax.jit, static_argnums=())
def _sc_embed(xflat, table):
    n = xflat.shape[0]
    per_worker = n // NUM_WORKERS
    num_batches = per_worker // BATCH
    mesh = plsc.VectorSubcoreMesh(core_axis_name="c", subcore_axis_name="s")

    @functools.partial(
        pl.kernel,
        mesh=mesh,
        compiler_params=pltpu.CompilerParams(use_tc_tiling_on_sc=False),
        out_type=jax.ShapeDtypeStruct((n, EMBED), jnp.float32),
        scratch_types=[
            pltpu.VMEM((per_worker,), jnp.int32),
            pltpu.VMEM((BATCH, EMBED), jnp.float32),
            pltpu.SemaphoreType.DMA,
        ],
    )
    def k(x_hbm, table_hbm, out_hbm, idx_v, rows_v, sem):
        wid = lax.axis_index("s") * 2 + lax.axis_index("c")
        base = wid * per_worker
        pltpu.sync_copy(x_hbm.at[pl.ds(base, per_worker)], idx_v)

        def mod_body(i, carry):
            o = i * jnp.int32(64)
            for j in range(4):
                v = idx_v[pl.ds(o + jnp.int32(j * 16), 16)]
                idx_v[pl.ds(o + jnp.int32(j * 16), 16)] = lax.rem(
                    v, jnp.int32(MOD))
            return carry

        lax.fori_loop(jnp.int32(0), jnp.int32(per_worker // 64), mod_body, 0)

        def batch_body(b, carry):
            off = b * jnp.int32(BATCH)
            cps = []
            for g in range(KB):
                cps.append(pltpu.async_copy(
                    table_hbm.at[idx_v.at[pl.ds(off + g * GATHER, GATHER)]],
                    rows_v.at[pl.ds(g * GATHER, GATHER)],
                    sem,
                ))
            for cp in cps:
                cp.wait()
            pltpu.sync_copy(rows_v, out_hbm.at[pl.ds(base + off, BATCH)])
            return carry

        lax.fori_loop(jnp.int32(0), jnp.int32(num_batches), batch_body, 0)

    return k(xflat, table)


def kernel(x, table):
    xflat = x.reshape(-1).astype(jnp.int32)
    out = _sc_embed(xflat, table)
    return out.reshape(x.shape[0], x.shape[1] * EMBED)


# speedup vs baseline: 2.0550x; 1.0049x over previous
import functools

import jax
import jax.numpy as jnp
from jax import lax
from jax.experimental import pallas as pl
from jax.experimental.pallas import tpu as pltpu
from jax.experimental.pallas import tpu_sc as plsc

MOD = 1000000
EMBED = 32
NUM_WORKERS = 32
GATHER = 128
KB = 4
BATCH = GATHER * KB
NBUF = 2


def _sc_embed(xflat, table):
    n = xflat.shape[0]
    per_worker = n // NUM_WORKERS
    num_outer = per_worker // (BATCH * NBUF)
    assert per_worker % (BATCH * NBUF) == 0
    mesh = plsc.VectorSubcoreMesh(core_axis_name="c", subcore_axis_name="s")

    @functools.partial(
        pl.kernel,
        mesh=mesh,
        compiler_params=pltpu.CompilerParams(use_tc_tiling_on_sc=False),
        out_type=jax.ShapeDtypeStruct((n, EMBED), jnp.float32),
        scratch_types=[
            pltpu.VMEM((per_worker,), jnp.int32),
            pltpu.VMEM((NBUF, BATCH, EMBED), jnp.float32),
            pltpu.SemaphoreType.DMA,
            pltpu.SemaphoreType.DMA,
            pltpu.SemaphoreType.DMA,
            pltpu.SemaphoreType.DMA,
        ],
    )
    def k(x_hbm, table_hbm, out_hbm, idx_v, rows_v, sg0, sg1, ss0, ss1):
        sem_g = (sg0, sg1)
        sem_s = (ss0, ss1)
        wid = lax.axis_index("s") * 2 + lax.axis_index("c")
        base = wid * per_worker
        pltpu.sync_copy(x_hbm.at[pl.ds(base, per_worker)], idx_v)

        def mod_body(i, carry):
            o = i * jnp.int32(64)
            for j in range(4):
                v = idx_v[pl.ds(o + jnp.int32(j * 16), 16)]
                idx_v[pl.ds(o + jnp.int32(j * 16), 16)] = lax.rem(
                    v, jnp.int32(MOD))
            return carry

        lax.fori_loop(jnp.int32(0), jnp.int32(per_worker // 64), mod_body, 0)

        def outer_body(t, carry):
            boff = t * jnp.int32(BATCH * NBUF)
            for p in range(NBUF):
                @pl.when(t > jnp.int32(0))
                def _():
                    pltpu.make_async_copy(
                        rows_v.at[jnp.int32(p)], out_hbm.at[pl.ds(base, BATCH)],
                        sem_s[p]).wait()
                off = boff + jnp.int32(p * BATCH)
                for g in range(KB):
                    pltpu.async_copy(
                        table_hbm.at[idx_v.at[pl.ds(off + jnp.int32(g * GATHER),
                                                    GATHER)]],
                        rows_v.at[jnp.int32(p)].at[pl.ds(g * GATHER, GATHER)],
                        sem_g[p],
                    )
            for p in range(NBUF):
                off = boff + jnp.int32(p * BATCH)
                pltpu.make_async_copy(
                    table_hbm.at[idx_v.at[pl.ds(off, GATHER)]],
                    rows_v.at[jnp.int32(p)], sem_g[p]).wait()
                pltpu.async_copy(rows_v.at[jnp.int32(p)],
                                 out_hbm.at[pl.ds(base + off, BATCH)],
                                 sem_s[p])
            return carry

        lax.fori_loop(jnp.int32(0), jnp.int32(num_outer), outer_body, 0)
        for p in range(NBUF):
            pltpu.make_async_copy(
                rows_v.at[jnp.int32(p)], out_hbm.at[pl.ds(base, BATCH)], sem_s[p]).wait()

    return k(xflat, table)


def kernel(x, table):
    xflat = x.reshape(-1).astype(jnp.int32)
    out = _sc_embed(xflat, table)
    return out.reshape(x.shape[0], x.shape[1] * EMBED)
